# baseline (speedup 1.0000x reference)
import functools

import jax
import jax.numpy as jnp
from jax import lax
from jax.experimental import pallas as pl
from jax.experimental.pallas import tpu as pltpu
from jax.experimental.pallas import tpu_sc as plsc

N_CELL = 400
W1 = N_CELL + 1
L = 16
B = 1024
NSTREAM = B // 128


def _make_sc_kernel(n_s, n_m, nc, ns):
    nw = nc * ns
    n_points = n_s * n_m
    pts_per_tile = n_points // nw
    tiles_per_row = n_m // pts_per_tile
    nchunks = pts_per_tile // B
    mesh = plsc.VectorSubcoreMesh(core_axis_name="c", subcore_axis_name="s")

    @functools.partial(
        pl.kernel,
        mesh=mesh,
        compiler_params=pltpu.CompilerParams(
            needs_layout_passes=False, use_tc_tiling_on_sc=False),
        out_type=jax.ShapeDtypeStruct((6, n_s // 8, n_m // 128, 8, 128),
                                      jnp.float32),
        scratch_types=[
            pltpu.VMEM((2, 8, 2, 128), jnp.float32),
            pltpu.VMEM((2, 8, 128), jnp.int32),
            pltpu.VMEM((2, B), jnp.float32),
            pltpu.VMEM((2, B), jnp.float32),
            pltpu.VMEM((2, B), jnp.float32),
            pltpu.VMEM((2, B, 32), jnp.float32),
            pltpu.VMEM((B, 33), jnp.float32),
            pltpu.VMEM((6, 8, 128), jnp.float32),
            pltpu.SemaphoreType.DMA,
            pltpu.SemaphoreType.DMA,
            pltpu.SemaphoreType.DMA,
        ],
    )
    def sc_kernel(xq_hbm, tab_hbm, out_hbm,
                  xyv, idxv, w00r, w10r, w01r, rows, rows33, outv,
                  xsem, gsem, osem):
        wid = lax.axis_index("s") * nc + lax.axis_index("c")
        s = wid // tiles_per_row
        s_hi = s // 8
        s_lo = s % 8
        m_base = (wid % tiles_per_row) * pts_per_tile
        viota = lax.iota(jnp.int32, L)

        def mt_of(c):
            return (m_base + c * B) // 128

        def xy_copy(c, buf):
            return pltpu.make_async_copy(
                xq_hbm.at[s, pl.ds(mt_of(c), 8)], xyv.at[buf], xsem)

        def gather_copies(buf):
            return [
                pltpu.make_async_copy(
                    tab_hbm.at[idxv.at[buf].at[j]],
                    rows.at[buf, pl.ds(j * 128, 128)],
                    gsem,
                )
                for j in range(NSTREAM)
            ]

        def out_copies(c):
            return [
                pltpu.make_async_copy(
                    outv.at[ch],
                    out_hbm.at[ch, s_hi, pl.ds(mt_of(c), 8), s_lo, :],
                    osem,
                )
                for ch in range(6)
            ]

        def phase1(buf):
            @plsc.parallel_loop(0, NSTREAM, unroll=4)
            def idx_body(j):
                for h in range(8):
                    g = j * 8 + h
                    xg = xyv[buf, j, 0, pl.ds(h * L, L)]
                    yg = xyv[buf, j, 1, pl.ds(h * L, L)]
                    ix = ((xg * 2.0 - 1.0 + 1.0) * N_CELL - 1.0) * 0.5
                    iy = ((yg * 2.0 - 1.0 + 1.0) * N_CELL - 1.0) * 0.5
                    fx = ix + 1.0
                    fy = iy + 1.0
                    jx = fx.astype(jnp.int32)
                    jy = fy.astype(jnp.int32)
                    wx1 = fx - jx.astype(jnp.float32)
                    wy1 = fy - jy.astype(jnp.float32)
                    wx0 = 1.0 - wx1
                    wy0 = 1.0 - wy1
                    idxv[buf, j, pl.ds(h * L, L)] = (jy * 402 + jx) * 4
                    off = g * L
                    w00r[buf, pl.ds(off, L)] = wx0 * wy0
                    w10r[buf, pl.ds(off, L)] = wx1 * wy0
                    w01r[buf, pl.ds(off, L)] = wx0 * wy1

        def phase3(buf):
            @plsc.parallel_loop(0, B // L, unroll=8)
            def grp_body(g):
                j = g // 8
                col = (g % 8) * L
                rbase = viota + g * L
                off = g * L
                w00 = w00r[buf, pl.ds(off, L)]
                w10 = w10r[buf, pl.ds(off, L)]
                w01 = w01r[buf, pl.ds(off, L)]
                w11 = ((1.0 - w00) - w10) - w01
                for l in range(L):
                    p = off + l
                    rows33[p, pl.ds(0, L)] = rows[buf, p, pl.ds(0, L)]
                    rows33[p, pl.ds(L, L)] = rows[buf, p, pl.ds(L, L)]
                ga = [plsc.load_gather(
                    rows33, [rbase, jnp.full((L,), ch, jnp.int32)])
                    for ch in range(6)]
                gb = [plsc.load_gather(
                    rows33, [rbase, jnp.full((L,), 8 + ch, jnp.int32)])
                    for ch in range(6)]
                gc = [plsc.load_gather(
                    rows33, [rbase, jnp.full((L,), 16 + ch, jnp.int32)])
                    for ch in range(6)]
                gd = [plsc.load_gather(
                    rows33, [rbase, jnp.full((L,), 24 + ch, jnp.int32)])
                    for ch in range(6)]
                t = [(w00 * ga[ch] + w10 * gb[ch])
                     + (w01 * gc[ch] + w11 * gd[ch]) for ch in range(6)]
                es = [jnp.exp(-t[ch]) for ch in range(3)]
                for ch in range(3):
                    t[ch] = 1.0 / (1.0 + es[ch])
                for ch in range(6):
                    outv[ch, j, pl.ds(col, L)] = t[ch]

        pltpu.sync_copy(xq_hbm.at[s, pl.ds(mt_of(0), 8)], xyv.at[0])
        phase1(0)
        for cp in gather_copies(0):
            cp.start()
        xy_copy(1, 1).start()

        def chunk_pair(cc, carry):
            for par in range(2):
                c = cc * 2 + par
                buf = par
                nb = 1 - par

                @pl.when(c + 1 < nchunks)
                def _():
                    xy_copy(c + 1, nb).wait()
                    phase1(nb)
                    for cp in gather_copies(nb):
                        cp.start()

                @pl.when(c + 2 < nchunks)
                def _():
                    xy_copy(c + 2, buf).start()

                for cp in gather_copies(buf):
                    cp.wait()

                @pl.when(c >= 1)
                def _():
                    for cp in out_copies(c - 1):
                        cp.wait()

                phase3(buf)
                for cp in out_copies(c):
                    cp.start()
            return carry

        lax.fori_loop(0, nchunks // 2, chunk_pair, 0)

        for cp in out_copies(nchunks - 1):
            cp.wait()

    return sc_kernel


def kernel(x, color, grid):
    n_s, n_m, _ = x.shape

    img = jnp.concatenate([color[0], grid[0]], axis=0)
    ip = jnp.pad(img, ((0, 2), (1, 1), (1, 1)))
    ip2 = ip.reshape(8, 402 * 402)
    nrow = 400 * 402 + 401
    corners = jnp.concatenate(
        [ip2[:, 0:nrow], ip2[:, 1:nrow + 1],
         ip2[:, 402:nrow + 402], ip2[:, 403:nrow + 403]],
        axis=0,
    )
    nrow_pad = (nrow + 7) // 8 * 8
    eye = jnp.eye(32, 128, dtype=jnp.float32)
    src = jnp.pad(corners, ((0, 0), (0, nrow_pad - nrow)))
    tab4 = jax.lax.dot_general(
        src, eye,
        dimension_numbers=(((0,), (0,)), ((), ())),
        preferred_element_type=jnp.float32,
        precision=lax.Precision.HIGH,
    )
    tab = tab4.reshape(nrow_pad * 4, 32)

    xq = x.reshape(n_s, n_m // 128, 128, 2).transpose(0, 1, 3, 2)

    info = plsc.get_sparse_core_info()
    sc_kernel = _make_sc_kernel(n_s, n_m, info.num_cores, info.num_subcores)
    out = sc_kernel(xq, tab)

    return out.transpose(1, 3, 2, 4, 0).reshape(n_s, n_m, 6)


# speedup vs baseline: 1.0039x; 1.0039x over previous
import functools

import jax
import jax.numpy as jnp
from jax import lax
from jax.experimental import pallas as pl
from jax.experimental.pallas import tpu as pltpu
from jax.experimental.pallas import tpu_sc as plsc

N_CELL = 400
L = 16
B = 1024
NSTREAM = B // 128


def _make_sc_kernel(n_s, n_m, nc, ns):
    nw = nc * ns
    n_points = n_s * n_m
    pts_per_tile = n_points // nw
    tiles_per_row = n_m // pts_per_tile
    nchunks = pts_per_tile // B
    mesh = plsc.VectorSubcoreMesh(core_axis_name="c", subcore_axis_name="s")

    @functools.partial(
        pl.kernel,
        mesh=mesh,
        compiler_params=pltpu.CompilerParams(
            needs_layout_passes=False, use_tc_tiling_on_sc=False),
        out_type=jax.ShapeDtypeStruct((6, n_s // 8, n_m // 128, 8, 128),
                                      jnp.float32),
        scratch_types=[
            pltpu.VMEM((2, 8, 2, 128), jnp.float32),
            pltpu.VMEM((2, 8, 128), jnp.int32),
            pltpu.VMEM((2, B), jnp.float32),
            pltpu.VMEM((2, B), jnp.float32),
            pltpu.VMEM((2, B), jnp.float32),
            pltpu.VMEM((2, B, 32), jnp.float32),
            pltpu.VMEM((B, 33), jnp.float32),
            pltpu.VMEM((6, 8, 128), jnp.float32),
            pltpu.SemaphoreType.DMA,
            pltpu.SemaphoreType.DMA,
            pltpu.SemaphoreType.DMA,
        ],
    )
    def sc_kernel(xq_hbm, tab_hbm, out_hbm,
                  xyv, idxv, w00r, w10r, w01r, rows, rows33, outv,
                  xsem, gsem, osem):
        wid = lax.axis_index("s") * nc + lax.axis_index("c")
        s = wid // tiles_per_row
        s_hi = s // 8
        s_lo = s % 8
        m_base = (wid % tiles_per_row) * pts_per_tile
        viota = lax.iota(jnp.int32, L)

        def mt_of(c):
            return (m_base + c * B) // 128

        def xy_copy(c, buf):
            return pltpu.make_async_copy(
                xq_hbm.at[s, pl.ds(mt_of(c), 8)], xyv.at[buf], xsem)

        def gather_copies(buf):
            return [
                pltpu.make_async_copy(
                    tab_hbm.at[idxv.at[buf].at[j]],
                    rows.at[buf, pl.ds(j * 128, 128)],
                    gsem,
                )
                for j in range(NSTREAM)
            ]

        def out_copies(c):
            return [
                pltpu.make_async_copy(
                    outv.at[ch],
                    out_hbm.at[ch, s_hi, pl.ds(mt_of(c), 8), s_lo, :],
                    osem,
                )
                for ch in range(6)
            ]

        def phase1(buf):
            @plsc.parallel_loop(0, NSTREAM, unroll=4)
            def idx_body(j):
                for h in range(8):
                    g = j * 8 + h
                    xg = xyv[buf, j, 0, pl.ds(h * L, L)]
                    yg = xyv[buf, j, 1, pl.ds(h * L, L)]
                    ix = ((xg * 2.0 - 1.0 + 1.0) * N_CELL - 1.0) * 0.5
                    iy = ((yg * 2.0 - 1.0 + 1.0) * N_CELL - 1.0) * 0.5
                    fx = ix + 1.0
                    fy = iy + 1.0
                    jx = fx.astype(jnp.int32)
                    jy = fy.astype(jnp.int32)
                    wx1 = fx - jx.astype(jnp.float32)
                    wy1 = fy - jy.astype(jnp.float32)
                    wx0 = 1.0 - wx1
                    wy0 = 1.0 - wy1
                    idxv[buf, j, pl.ds(h * L, L)] = (jy * 402 + jx) * 4
                    off = g * L
                    w00r[buf, pl.ds(off, L)] = wx0 * wy0
                    w10r[buf, pl.ds(off, L)] = wx1 * wy0
                    w01r[buf, pl.ds(off, L)] = wx0 * wy1

        def phase3(buf):
            @plsc.parallel_loop(0, B // L, unroll=8)
            def grp_body(g):
                j = g // 8
                col = (g % 8) * L
                rbase = viota + g * L
                off = g * L
                w00 = w00r[buf, pl.ds(off, L)]
                w10 = w10r[buf, pl.ds(off, L)]
                w01 = w01r[buf, pl.ds(off, L)]
                w11 = ((1.0 - w00) - w10) - w01
                for l in range(L):
                    p = off + l
                    rows33[p, pl.ds(0, L)] = rows[buf, p, pl.ds(0, L)]
                    rows33[p, pl.ds(L, L)] = rows[buf, p, pl.ds(L, L)]
                ga = [plsc.load_gather(
                    rows33, [rbase, jnp.full((L,), ch, jnp.int32)])
                    for ch in range(6)]
                gb = [plsc.load_gather(
                    rows33, [rbase, jnp.full((L,), 8 + ch, jnp.int32)])
                    for ch in range(6)]
                gc = [plsc.load_gather(
                    rows33, [rbase, jnp.full((L,), 16 + ch, jnp.int32)])
                    for ch in range(6)]
                gd = [plsc.load_gather(
                    rows33, [rbase, jnp.full((L,), 24 + ch, jnp.int32)])
                    for ch in range(6)]
                t = [(w00 * ga[ch] + w10 * gb[ch])
                     + (w01 * gc[ch] + w11 * gd[ch]) for ch in range(6)]
                es = [jnp.exp(-t[ch]) for ch in range(3)]
                for ch in range(3):
                    t[ch] = 1.0 / (1.0 + es[ch])
                for ch in range(6):
                    outv[ch, j, pl.ds(col, L)] = t[ch]

        pltpu.sync_copy(xq_hbm.at[s, pl.ds(mt_of(0), 8)], xyv.at[0])
        phase1(0)
        for cp in gather_copies(0):
            cp.start()
        xy_copy(1, 1).start()

        def chunk_pair(cc, carry):
            for par in range(2):
                c = cc * 2 + par
                buf = par
                nb = 1 - par

                @pl.when(c + 1 < nchunks)
                def _():
                    xy_copy(c + 1, nb).wait()
                    phase1(nb)
                    for cp in gather_copies(nb):
                        cp.start()

                @pl.when(c + 2 < nchunks)
                def _():
                    xy_copy(c + 2, buf).start()

                for cp in gather_copies(buf):
                    cp.wait()

                @pl.when(c >= 1)
                def _():
                    for cp in out_copies(c - 1):
                        cp.wait()

                phase3(buf)
                for cp in out_copies(c):
                    cp.start()
            return carry

        lax.fori_loop(0, nchunks // 2, chunk_pair, 0)

        for cp in out_copies(nchunks - 1):
            cp.wait()

    return sc_kernel


def kernel(x, color, grid):
    n_s, n_m, _ = x.shape

    img = jnp.concatenate([color[0], grid[0]], axis=0)
    ip = jnp.pad(img, ((0, 2), (1, 1), (1, 1)))
    ip2 = ip.reshape(8, 402 * 402)
    nrow = 400 * 402 + 401
    corners = jnp.concatenate(
        [ip2[:, 0:nrow], ip2[:, 1:nrow + 1],
         ip2[:, 402:nrow + 402], ip2[:, 403:nrow + 403]],
        axis=0,
    )
    nrow_pad = (nrow + 7) // 8 * 8
    eye = jnp.eye(32, 128, dtype=jnp.float32)
    src = jnp.pad(corners, ((0, 0), (0, nrow_pad - nrow)))
    tab4 = jax.lax.dot_general(
        src, eye,
        dimension_numbers=(((0,), (0,)), ((), ())),
        preferred_element_type=jnp.float32,
        precision=lax.Precision.HIGH,
    )
    tab = tab4.reshape(nrow_pad * 4, 32)

    xq = x.reshape(n_s, n_m // 128, 128, 2).transpose(0, 1, 3, 2)

    info = plsc.get_sparse_core_info()
    sc_kernel = _make_sc_kernel(n_s, n_m, info.num_cores, info.num_subcores)
    out = sc_kernel(xq, tab)

    return out.transpose(1, 3, 2, 4, 0).reshape(n_s, n_m, 6)
